# baseline (speedup 1.0000x reference)
import functools

import jax
import jax.numpy as jnp
from jax.experimental import pallas as pl

NUM_QUANTIZERS = 6
NB_CODE = 1024
CODE_DIM = 128
TILE = 1024


def _rvq_stage_kernel(z_ref, cb_ref, out_ref, codes_ref):
    zb = z_ref[...]
    residual = zb
    quantized = jnp.zeros_like(zb)
    for q in range(NUM_QUANTIZERS):
        cb = cb_ref[q]
        c2 = jnp.sum(cb * cb, axis=1)
        rr = jnp.sum(residual * residual, axis=1, keepdims=True)
        rc = jax.lax.dot_general(
            residual, cb,
            dimension_numbers=(((1,), (1,)), ((), ())),
            preferred_element_type=jnp.float32,
        )
        dist = rr - 2.0 * rc + c2[None, :]
        idx = jnp.argmin(dist, axis=1).astype(jnp.int32)
        onehot = (
            idx[:, None]
            == jax.lax.broadcasted_iota(jnp.int32, (TILE, NB_CODE), 1)
        ).astype(jnp.float32)
        qv = jax.lax.dot_general(
            onehot, cb,
            dimension_numbers=(((1,), (0,)), ((), ())),
            precision=jax.lax.Precision.HIGHEST,
            preferred_element_type=jnp.float32,
        )
        quantized = quantized + qv
        residual = residual - qv
        codes_ref[q, :] = idx
    out_ref[...] = zb + (quantized - zb)


@functools.partial(jax.jit, static_argnames=())
def kernel(z, codebooks):
    b, t, d = z.shape
    n_tok = b * t
    flat = z.reshape(n_tok, d)
    n_tiles = n_tok // TILE

    out_flat, codes_raw = pl.pallas_call(
        _rvq_stage_kernel,
        grid=(n_tiles,),
        in_specs=[
            pl.BlockSpec((TILE, d), lambda i: (i, 0)),
            pl.BlockSpec(
                (NUM_QUANTIZERS, NB_CODE, d), lambda i: (0, 0, 0)
            ),
        ],
        out_specs=[
            pl.BlockSpec((TILE, d), lambda i: (i, 0)),
            pl.BlockSpec((8, TILE), lambda i: (0, i)),
        ],
        out_shape=[
            jax.ShapeDtypeStruct((n_tok, d), jnp.float32),
            jax.ShapeDtypeStruct((8, n_tok), jnp.int32),
        ],
    )(flat, codebooks)

    out = out_flat.reshape(b, t, d)
    codes = codes_raw[:NUM_QUANTIZERS].reshape(NUM_QUANTIZERS, b, t)
    return out, codes


# speedup vs baseline: 1.9900x; 1.9900x over previous
import jax
import jax.numpy as jnp
from jax.experimental import pallas as pl

NUM_QUANTIZERS = 6
NB_CODE = 1024
CODE_DIM = 128
TILE = 1024


def _rvq_kernel(z_ref, cb_ref, hi_ref, mid_ref, lo_ref, out_ref, codes_ref):
    zb = z_ref[...]
    residual = zb
    quantized = jnp.zeros_like(zb)
    for q in range(NUM_QUANTIZERS):
        cb = cb_ref[q]
        c2 = jnp.sum(cb * cb, axis=1)
        rr = jnp.sum(residual * residual, axis=1, keepdims=True)
        rc = jax.lax.dot_general(
            residual, cb,
            dimension_numbers=(((1,), (1,)), ((), ())),
            preferred_element_type=jnp.float32,
        )
        dist = rr - 2.0 * rc + c2[None, :]
        idx = jnp.argmin(dist, axis=1).astype(jnp.int32)
        onehot = (
            idx[:, None]
            == jax.lax.broadcasted_iota(jnp.int32, (TILE, NB_CODE), 1)
        ).astype(jnp.bfloat16)
        qv_hi = jax.lax.dot_general(
            onehot, hi_ref[q],
            dimension_numbers=(((1,), (0,)), ((), ())),
            preferred_element_type=jnp.float32,
        )
        qv_mid = jax.lax.dot_general(
            onehot, mid_ref[q],
            dimension_numbers=(((1,), (0,)), ((), ())),
            preferred_element_type=jnp.float32,
        )
        qv_lo = jax.lax.dot_general(
            onehot, lo_ref[q],
            dimension_numbers=(((1,), (0,)), ((), ())),
            preferred_element_type=jnp.float32,
        )
        qv = (qv_hi + qv_mid) + qv_lo
        quantized = quantized + qv
        residual = residual - qv
        codes_ref[q, :] = idx
    out_ref[...] = zb + (quantized - zb)


def kernel(z, codebooks):
    b, t, d = z.shape
    n_tok = b * t
    flat = z.reshape(n_tok, d)
    n_tiles = n_tok // TILE

    cb_hi = codebooks.astype(jnp.bfloat16)
    r1 = codebooks - cb_hi.astype(jnp.float32)
    cb_mid = r1.astype(jnp.bfloat16)
    cb_lo = (r1 - cb_mid.astype(jnp.float32)).astype(jnp.bfloat16)

    cb_spec = pl.BlockSpec((NUM_QUANTIZERS, NB_CODE, d), lambda i: (0, 0, 0))
    out_flat, codes_raw = pl.pallas_call(
        _rvq_kernel,
        grid=(n_tiles,),
        in_specs=[
            pl.BlockSpec((TILE, d), lambda i: (i, 0)),
            cb_spec, cb_spec, cb_spec, cb_spec,
        ],
        out_specs=[
            pl.BlockSpec((TILE, d), lambda i: (i, 0)),
            pl.BlockSpec((8, TILE), lambda i: (0, i)),
        ],
        out_shape=[
            jax.ShapeDtypeStruct((n_tok, d), jnp.float32),
            jax.ShapeDtypeStruct((8, n_tok), jnp.int32),
        ],
    )(flat, codebooks, cb_hi, cb_mid, cb_lo)

    out = out_flat.reshape(b, t, d)
    codes = codes_raw[:NUM_QUANTIZERS].reshape(NUM_QUANTIZERS, b, t)
    return out, codes
